# baseline (speedup 1.0000x reference)
import functools

import jax
import jax.numpy as jnp
from jax.experimental import pallas as pl
from jax.experimental.pallas import tpu as pltpu

K = 16
Q = 512


def _knn_body(q_ref, kt_ref, k_ref, o_ref):
    keys_t = kt_ref[0]
    keys = k_ref[0]
    n = keys.shape[0]
    queries_t = q_ref[0].T

    rows = []
    for q in range(Q):
        dq = jnp.abs(queries_t[:, q:q + 1] - keys_t)
        rows.append(jnp.sum(dq, axis=0, keepdims=True))
    dist = jnp.concatenate(rows, axis=0)

    col = jax.lax.broadcasted_iota(jnp.int32, dist.shape, 1)

    for j in range(K + 1):
        mn = jnp.min(dist, axis=1, keepdims=True)
        eq = dist == mn
        idxv = jnp.min(jnp.where(eq, col, n), axis=1)
        sel = col == idxv[:, None]
        dist = jnp.where(sel, jnp.inf, dist)
        nb = jnp.dot(sel.astype(jnp.float32), keys,
                     preferred_element_type=jnp.float32,
                     precision=jax.lax.Precision.HIGHEST)
        o_ref[0, :, j, :] = nb


def kernel(inputs):
    B, N, F = inputs.shape
    inputs_t = jnp.transpose(inputs, (0, 2, 1))
    grid = (B, N // Q)
    out = pl.pallas_call(
        _knn_body,
        grid=grid,
        in_specs=[
            pl.BlockSpec((1, Q, F), lambda b, nb: (b, nb, 0)),
            pl.BlockSpec((1, F, N), lambda b, nb: (b, 0, 0)),
            pl.BlockSpec((1, N, F), lambda b, nb: (b, 0, 0)),
        ],
        out_specs=pl.BlockSpec((1, Q, K + 1, F), lambda b, nb: (b, nb, 0, 0)),
        out_shape=jax.ShapeDtypeStruct((B, N, K + 1, F), jnp.float32),
        compiler_params=pltpu.CompilerParams(
            dimension_semantics=("parallel", "arbitrary"),
        ),
    )(inputs, inputs_t, inputs)
    return jnp.transpose(out, (0, 1, 3, 2))


# speedup vs baseline: 1.3363x; 1.3363x over previous
import functools

import jax
import jax.numpy as jnp
from jax.experimental import pallas as pl
from jax.experimental.pallas import tpu as pltpu

K = 16
Q = 256


def _knn_body(q_ref, kt_ref, k_ref, o_ref):
    keys_t = kt_ref[0]
    keys = k_ref[0]
    n = keys.shape[0]
    queries_t = q_ref[0].T

    f = keys_t.shape[0]
    rows = []
    for q in range(Q):
        qc = queries_t[:, q:q + 1]
        acc = jnp.abs(qc[0:8] - keys_t[0:8])
        for r in range(8, f, 8):
            acc = acc + jnp.abs(qc[r:r + 8] - keys_t[r:r + 8])
        t4 = acc[0:4] + acc[4:8]
        t2 = t4[0:2] + t4[2:4]
        rows.append(t2[0:1] + t2[1:2])
    dist = jnp.concatenate(rows, axis=0)

    col = jax.lax.broadcasted_iota(jnp.int32, dist.shape, 1)

    for j in range(K + 1):
        mn = jnp.min(dist, axis=1, keepdims=True)
        eq = dist == mn
        idxv = jnp.min(jnp.where(eq, col, n), axis=1)
        sel = col == idxv[:, None]
        dist = jnp.where(sel, jnp.inf, dist)
        nb = jnp.dot(sel.astype(jnp.float32), keys,
                     preferred_element_type=jnp.float32,
                     precision=jax.lax.Precision.HIGHEST)
        o_ref[0, :, j, :] = nb


def kernel(inputs):
    B, N, F = inputs.shape
    inputs_t = jnp.transpose(inputs, (0, 2, 1))
    grid = (B, N // Q)
    out = pl.pallas_call(
        _knn_body,
        grid=grid,
        in_specs=[
            pl.BlockSpec((1, Q, F), lambda b, nb: (b, nb, 0)),
            pl.BlockSpec((1, F, N), lambda b, nb: (b, 0, 0)),
            pl.BlockSpec((1, N, F), lambda b, nb: (b, 0, 0)),
        ],
        out_specs=pl.BlockSpec((1, Q, K + 1, F), lambda b, nb: (b, nb, 0, 0)),
        out_shape=jax.ShapeDtypeStruct((B, N, K + 1, F), jnp.float32),
        compiler_params=pltpu.CompilerParams(
            dimension_semantics=("parallel", "arbitrary"),
        ),
    )(inputs, inputs_t, inputs)
    return jnp.transpose(out, (0, 1, 3, 2))
